# baseline (speedup 1.0000x reference)
import functools

import jax
import jax.numpy as jnp
from jax import lax
from jax.experimental import pallas as pl
from jax.experimental.pallas import tpu as pltpu
from jax.experimental.pallas import tpu_sc as plsc

B = 4096
V = 1000000
DIN = 64
DITEM = 128
NNEG = 20

NC = 2
NS = 16
NW = NC * NS
PB = B // NW
NB = PB * NNEG
NEG_CHUNK = 640
NUM_NEG_CHUNKS = NB // NEG_CHUNK

_sc_mesh = plsc.VectorSubcoreMesh(core_axis_name="c", subcore_axis_name="s")


@functools.partial(
    pl.kernel,
    out_type=[
        jax.ShapeDtypeStruct((B, DIN), jnp.float32),
        jax.ShapeDtypeStruct((B, DITEM), jnp.float32),
        jax.ShapeDtypeStruct((B * NNEG, DITEM), jnp.float32),
    ],
    mesh=_sc_mesh,
    compiler_params=pltpu.CompilerParams(use_tc_tiling_on_sc=False),
    scratch_types=[
        pltpu.VMEM((PB,), jnp.int32),
        pltpu.VMEM((PB,), jnp.int32),
        pltpu.VMEM((NB,), jnp.int32),
        pltpu.VMEM((PB, DIN), jnp.float32),
        pltpu.VMEM((PB, DITEM), jnp.float32),
        pltpu.VMEM((NEG_CHUNK, DITEM), jnp.float32),
        pltpu.SemaphoreType.DMA,
    ],
)
def _sc_gather(pos_input_hbm, pos_item_hbm, neg_hbm, input_emb_hbm,
               item_emb_hbm, out_in_hbm, out_item_hbm, out_neg_hbm,
               idx_in_v, idx_item_v, idx_neg_v, rows_in_v, rows_item_v,
               rows_neg_v, sem):
    wid = lax.axis_index("s") * NC + lax.axis_index("c")
    base = wid * PB
    nbase = wid * NB
    pltpu.sync_copy(pos_input_hbm.at[pl.ds(base, PB)], idx_in_v)
    pltpu.sync_copy(pos_item_hbm.at[pl.ds(base, PB)], idx_item_v)
    pltpu.sync_copy(neg_hbm.at[pl.ds(nbase, NB)], idx_neg_v)
    pltpu.async_copy(input_emb_hbm.at[idx_in_v], rows_in_v, sem).wait()
    pltpu.sync_copy(rows_in_v, out_in_hbm.at[pl.ds(base, PB)])
    pltpu.async_copy(item_emb_hbm.at[idx_item_v], rows_item_v, sem).wait()
    pltpu.sync_copy(rows_item_v, out_item_hbm.at[pl.ds(base, PB)])
    for c in range(NUM_NEG_CHUNKS):
        idx_slice = idx_neg_v.at[pl.ds(c * NEG_CHUNK, NEG_CHUNK)]
        pltpu.async_copy(item_emb_hbm.at[idx_slice], rows_neg_v, sem).wait()
        pltpu.sync_copy(rows_neg_v,
                        out_neg_hbm.at[pl.ds(nbase + c * NEG_CHUNK, NEG_CHUNK)])


TC_BLOCK = 512
TC_GRID = B // TC_BLOCK


def _tc_body(emb_in_ref, item_ref, neg_ref, w1_ref, b1_ref, w2_ref, b2_ref,
             out_ref):
    x = emb_in_ref[...]
    h = jnp.maximum(
        jnp.dot(x, w1_ref[...].T, preferred_element_type=jnp.float32)
        + b1_ref[...], 0.0)
    u = (jnp.dot(h, w2_ref[...].T, preferred_element_type=jnp.float32)
         + b2_ref[...])
    s = jnp.sum(u * item_ref[...], axis=1)
    s = jnp.clip(s, -10.0, 10.0)
    pos_loss = jnp.log1p(jnp.exp(-s))
    neg = neg_ref[...].reshape(TC_BLOCK, NNEG, DITEM)
    ns = jnp.sum(neg * u[:, None, :], axis=2)
    ns = jnp.clip(ns, -10.0, 10.0)
    neg_loss = jnp.sum(jnp.log1p(jnp.exp(ns)), axis=1)
    partial = (jnp.sum(pos_loss + neg_loss) * (1.0 / B)).reshape(1, 1)

    @pl.when(pl.program_id(0) == 0)
    def _init():
        out_ref[...] = partial

    @pl.when(pl.program_id(0) > 0)
    def _acc():
        out_ref[...] += partial


_tc_loss = pl.pallas_call(
    _tc_body,
    grid=(TC_GRID,),
    in_specs=[
        pl.BlockSpec((TC_BLOCK, DIN), lambda g: (g, 0)),
        pl.BlockSpec((TC_BLOCK, DITEM), lambda g: (g, 0)),
        pl.BlockSpec((TC_BLOCK * NNEG, DITEM), lambda g: (g, 0)),
        pl.BlockSpec((512, DIN), lambda g: (0, 0)),
        pl.BlockSpec((1, 512), lambda g: (0, 0)),
        pl.BlockSpec((DITEM, 512), lambda g: (0, 0)),
        pl.BlockSpec((1, DITEM), lambda g: (0, 0)),
    ],
    out_specs=pl.BlockSpec((1, 1), lambda g: (0, 0)),
    out_shape=jax.ShapeDtypeStruct((1, 1), jnp.float32),
)


def kernel(pos_input, pos_item, neg_item, i, input_emb, item_emb, W1, b1, W2,
           b2):
    del i
    pos_input = pos_input.astype(jnp.int32)
    pos_item = pos_item.astype(jnp.int32)
    neg_flat = neg_item.reshape(B * NNEG).astype(jnp.int32)
    emb_in, emb_item, emb_neg = _sc_gather(pos_input, pos_item, neg_flat,
                                           input_emb, item_emb)
    out = _tc_loss(emb_in, emb_item, emb_neg, W1, b1.reshape(1, 512), W2,
                   b2.reshape(1, DITEM))
    return out.reshape(())


# speedup vs baseline: 2.1011x; 2.1011x over previous
import functools

import jax
import jax.numpy as jnp
from jax import lax
from jax.experimental import pallas as pl
from jax.experimental.pallas import tpu as pltpu
from jax.experimental.pallas import tpu_sc as plsc

B = 4096
V = 1000000
DIN = 64
DITEM = 128
NNEG = 20

NC = 2
NS = 16
NW = NC * NS
PB = B // NW
NB = PB * NNEG
CHUNK_B = 16
CHUNK_ROWS = CHUNK_B * NNEG
NUM_CHUNKS = PB // CHUNK_B

_sc_mesh = plsc.VectorSubcoreMesh(core_axis_name="c", subcore_axis_name="s")


def _tc_mlp_body(x_ref, w1t_ref, b1_ref, w2t_ref, b2_ref, out_ref):
    h = jnp.maximum(
        jnp.dot(x_ref[...], w1t_ref[...],
                preferred_element_type=jnp.float32) + b1_ref[...],
        0.0)
    u = (jnp.dot(h, w2t_ref[...], preferred_element_type=jnp.float32)
         + b2_ref[...])
    out_ref[...] = u


_tc_mlp = pl.pallas_call(
    _tc_mlp_body,
    out_shape=jax.ShapeDtypeStruct((B, DITEM), jnp.float32),
)


@functools.partial(
    pl.kernel,
    out_type=[
        jax.ShapeDtypeStruct((B, DITEM), jnp.float32),
        jax.ShapeDtypeStruct((B, DITEM), jnp.float32),
    ],
    mesh=_sc_mesh,
    compiler_params=pltpu.CompilerParams(needs_layout_passes=False),
    scratch_types=[
        pltpu.VMEM((PB,), jnp.int32),
        pltpu.VMEM((NB,), jnp.int32),
        pltpu.VMEM((PB, DITEM), jnp.float32),
        pltpu.VMEM((PB, DITEM), jnp.float32),
        pltpu.VMEM((CHUNK_ROWS, DITEM), jnp.float32),
        pltpu.VMEM((CHUNK_ROWS, DITEM), jnp.float32),
        pltpu.VMEM((32, 16), jnp.float32),
        pltpu.VMEM((CHUNK_B, DITEM), jnp.float32),
        pltpu.SemaphoreType.DMA,
        pltpu.SemaphoreType.DMA,
        pltpu.SemaphoreType.DMA,
    ],
)
def _sc_stage_c(pos_item_hbm, neg_idx_hbm, item_emb_hbm, emb_user_hbm,
                out_item_hbm, out_scores_hbm,
                idx_item_v, idx_v, rows_item_v, user_v, buf0, buf1, red_v,
                scores_v, sem0, sem1, sem_i):
    wid = lax.axis_index("s") * NC + lax.axis_index("c")
    base = wid * PB
    nbase = wid * NB
    pltpu.sync_copy(pos_item_hbm.at[pl.ds(base, PB)], idx_item_v)
    pltpu.sync_copy(neg_idx_hbm.at[pl.ds(nbase, NB)], idx_v)
    item_cp = pltpu.async_copy(item_emb_hbm.at[idx_item_v], rows_item_v,
                               sem_i)
    pltpu.sync_copy(emb_user_hbm.at[pl.ds(base, PB)], user_v)

    bufs = (buf0, buf1)
    sems = (sem0, sem1)

    def start(c):
        idx_slice = idx_v.at[pl.ds(c * CHUNK_ROWS, CHUNK_ROWS)]
        return pltpu.async_copy(item_emb_hbm.at[idx_slice], bufs[c % 2],
                                sems[c % 2])

    iota16 = lax.iota(jnp.int32, 16)
    pending = start(0)
    for c in range(NUM_CHUNKS):
        pending.wait()
        if c + 1 < NUM_CHUNKS:
            pending = start(c + 1)
        buf = bufs[c % 2]

        @pl.loop(0, CHUNK_B)
        def _batch_loop(j):
            row = c * CHUNK_B + j
            uv = [user_v[row, pl.ds(16 * k, 16)] for k in range(8)]
            for n in range(NNEG):
                r = j * NNEG + n
                acc = buf[r, pl.ds(0, 16)] * uv[0]
                for k in range(1, 8):
                    acc = acc + buf[r, pl.ds(16 * k, 16)] * uv[k]
                red_v[n, :] = acc
            res1 = plsc.load_gather(red_v, [iota16,
                                            jnp.zeros((16,), jnp.int32)])
            res2 = plsc.load_gather(red_v, [iota16 + 16,
                                            jnp.zeros((16,), jnp.int32)])
            for col in range(1, 16):
                cidx = jnp.full((16,), col, jnp.int32)
                res1 = res1 + plsc.load_gather(red_v, [iota16, cidx])
                res2 = res2 + plsc.load_gather(red_v, [iota16 + 16, cidx])
            scores_v[j, pl.ds(0, 16)] = res1
            scores_v[j, pl.ds(16, 16)] = res2

        pltpu.sync_copy(scores_v,
                        out_scores_hbm.at[pl.ds(base + c * CHUNK_B, CHUNK_B)])

    item_cp.wait()
    pltpu.sync_copy(rows_item_v, out_item_hbm.at[pl.ds(base, PB)])


def _tc_loss_body(user_ref, item_ref, scores_ref, out_ref):
    u = user_ref[...]
    s = jnp.sum(u * item_ref[...], axis=1)
    s = jnp.clip(s, -10.0, 10.0)
    pos_loss = jnp.log1p(jnp.exp(-s))
    ns = scores_ref[...][:, :NNEG]
    ns = jnp.clip(ns, -10.0, 10.0)
    neg_loss = jnp.sum(jnp.log1p(jnp.exp(ns)), axis=1)
    out_ref[...] = (jnp.sum(pos_loss + neg_loss) * (1.0 / B)).reshape(1, 1)


_tc_loss = pl.pallas_call(
    _tc_loss_body,
    out_shape=jax.ShapeDtypeStruct((1, 1), jnp.float32),
)


def kernel(pos_input, pos_item, neg_item, i, input_emb, item_emb, W1, b1, W2,
           b2):
    del i
    pos_input = pos_input.astype(jnp.int32)
    pos_item = pos_item.astype(jnp.int32)
    neg_flat = neg_item.reshape(B * NNEG).astype(jnp.int32)
    emb_in = jnp.take(input_emb, pos_input, axis=0)
    emb_user = _tc_mlp(emb_in, W1.T, b1.reshape(1, 512), W2.T,
                       b2.reshape(1, DITEM))
    emb_item, scores = _sc_stage_c(pos_item, neg_flat, item_emb, emb_user)
    out = _tc_loss(emb_user, emb_item, scores)
    return out.reshape(())
